# baseline (speedup 1.0000x reference)
import jax
import jax.numpy as jnp
from jax import lax
from jax.experimental import pallas as pl
from jax.experimental.pallas import tpu as pltpu

B, H, W = 2, 224, 224
K, D = 19, 1024
P = H * W
DT = 64
KPAD = 32


def _body(src_ref, colors_ref, table_ref, out_ref, onehot_ref):
    @pl.when(pl.program_id(1) == 0)
    def _compute_onehot():
        q = (src_ref[0] * 127.5 + 127.5).astype(jnp.int32)
        match = None
        for c in range(3):
            eq = q[c:c + 1, :] == colors_ref[:, c:c + 1]
            match = eq if match is None else (match & eq)
        kvec = lax.broadcasted_iota(jnp.int32, (K, P), 0)
        cls = jnp.min(jnp.where(match, kvec, KPAD - 1), axis=0, keepdims=True)
        onehot_ref[...] = (
            cls == lax.broadcasted_iota(jnp.int32, (KPAD, P), 0)
        ).astype(jnp.float32)

    out_ref[0] = lax.dot_general(
        table_ref[...], onehot_ref[...],
        (((1,), (0,)), ((), ())), preferred_element_type=jnp.float32)


def kernel(src, colors, feats):
    src_flat = src.reshape(B, 3, P)
    colors_i = colors.astype(jnp.int32)
    table = jnp.zeros((D, KPAD), jnp.float32).at[:, :K].set(feats.T)
    out = pl.pallas_call(
        _body,
        grid=(B, D // DT),
        in_specs=[
            pl.BlockSpec((1, 3, P), lambda b, j: (b, 0, 0)),
            pl.BlockSpec((K, 3), lambda b, j: (0, 0)),
            pl.BlockSpec((DT, KPAD), lambda b, j: (j, 0)),
        ],
        out_specs=pl.BlockSpec((1, DT, P), lambda b, j: (b, j, 0)),
        out_shape=jax.ShapeDtypeStruct((B, D, P), jnp.float32),
        scratch_shapes=[pltpu.VMEM((KPAD, P), jnp.float32)],
        compiler_params=pltpu.CompilerParams(
            dimension_semantics=("arbitrary", "arbitrary")),
    )(src_flat, colors_i, table)
    return out.reshape(B, D, H, W)
